# baseline (speedup 1.0000x reference)
import jax
import jax.numpy as jnp
from jax import lax
from jax.experimental import pallas as pl
from jax.experimental.pallas import tpu as pltpu
from jax.experimental.pallas import tpu_sc as plsc

B = 16384
F = 26
V = 100001
D = 32

_info = plsc.get_sparse_core_info()
NC, NS = _info.num_cores, _info.num_subcores
NW = NC * NS
HALF = B // 2


CHUNK = 2048
NCHUNK = B // CHUNK
UNROLL = 8
IQ = 4096
NQ = B // IQ
NSTAGE = 16


def _body(cat_hbm, tab_hbm, out_hbm, tv, idx_v, out_v, idx_sh,
          tsem, isem0, isem1, osem0, osem1):
    s = lax.axis_index("s")
    d = s * NC + lax.axis_index("c")
    osems = [osem0, osem1]
    isems = [isem0, isem1]

    pltpu.sync_copy(cat_hbm.at[s], idx_sh.at[s])
    plsc.subcore_barrier()

    def fire_iq(f, q):
        ib = q % 2

        @pl.when(f < NSTAGE)
        def _():
            pltpu.async_copy(
                idx_sh.at[f, pl.ds(q * IQ, IQ)], idx_v.at[ib], isems[ib])

        @pl.when(f >= NSTAGE)
        def _():
            pltpu.async_copy(
                cat_hbm.at[f, pl.ds(q * IQ, IQ)], idx_v.at[ib], isems[ib])

    def drain_iq(f, q):
        ib = q % 2
        pltpu.make_async_copy(
            cat_hbm.at[f, pl.ds(0, IQ)], idx_v.at[ib], isems[ib]).wait()

    def do_field(fi, carry):
        f = lax.rem(fi + d, F)
        tcp = pltpu.async_copy(tab_hbm.at[f, d], tv, tsem)
        fire_iq(f, 0)
        tcp.wait()
        c = f * D + d
        cps = [None, None]
        for j in range(NCHUNK):
            q, jq = divmod(j, NCHUNK // NQ)
            obuf = j % 2
            ib = q % 2
            if jq == 0:
                if q + 1 < NQ:
                    fire_iq(f, q + 1)
                drain_iq(f, q)
            if cps[obuf] is not None:
                cps[obuf].wait()

            @plsc.parallel_loop(0, CHUNK // 16, unroll=UNROLL)
            def gath(i):
                vidx = idx_v[ib, pl.ds(jq * CHUNK + i * 16, 16)]
                out_v[obuf, pl.ds(i * 16, 16)] = plsc.load_gather(tv, [vidx])
            cps[obuf] = pltpu.async_copy(
                out_v.at[obuf], out_hbm.at[c, pl.ds(j * CHUNK, CHUNK)],
                osems[obuf])
        cps[0].wait()
        cps[1].wait()
        return carry

    lax.fori_loop(0, F, do_field, 0)


def kernel(categorical_features, tables):
    catT = categorical_features.T
    tabT = tables.transpose(0, 2, 1)
    mesh = plsc.VectorSubcoreMesh(core_axis_name="c", subcore_axis_name="s")
    outT = pl.kernel(
        _body,
        mesh=mesh,
        compiler_params=pltpu.CompilerParams(needs_layout_passes=False),
        out_type=jax.ShapeDtypeStruct((F * D, B), jnp.float32),
        scratch_types=[
            pltpu.VMEM((V,), jnp.float32),
            pltpu.VMEM((2, IQ), jnp.int32),
            pltpu.VMEM((2, CHUNK), jnp.float32),
            pltpu.VMEM_SHARED((NSTAGE, B), jnp.int32),
            pltpu.SemaphoreType.DMA,
            pltpu.SemaphoreType.DMA,
            pltpu.SemaphoreType.DMA,
            pltpu.SemaphoreType.DMA,
            pltpu.SemaphoreType.DMA,
        ],
    )(catT, tabT)
    return outT.T


# speedup vs baseline: 1.0000x; 1.0000x over previous
import jax
import jax.numpy as jnp
from jax import lax
from jax.experimental import pallas as pl
from jax.experimental.pallas import tpu as pltpu
from jax.experimental.pallas import tpu_sc as plsc

B = 16384
F = 26
V = 100001
D = 32

_info = plsc.get_sparse_core_info()
NC, NS = _info.num_cores, _info.num_subcores
NW = NC * NS
HALF = B // 2


CHUNK = 2048
NCHUNK = B // CHUNK
UNROLL = 16
IQ = 4096
NQ = B // IQ
NSTAGE = 16


def _body(cat_hbm, tab_hbm, out_hbm, tv, idx_v, out_v, idx_sh,
          tsem, isem0, isem1, osem0, osem1):
    s = lax.axis_index("s")
    d = s * NC + lax.axis_index("c")
    osems = [osem0, osem1]
    isems = [isem0, isem1]

    pltpu.sync_copy(cat_hbm.at[s], idx_sh.at[s])
    plsc.subcore_barrier()

    def fire_iq(f, q):
        ib = q % 2

        @pl.when(f < NSTAGE)
        def _():
            pltpu.async_copy(
                idx_sh.at[f, pl.ds(q * IQ, IQ)], idx_v.at[ib], isems[ib])

        @pl.when(f >= NSTAGE)
        def _():
            pltpu.async_copy(
                cat_hbm.at[f, pl.ds(q * IQ, IQ)], idx_v.at[ib], isems[ib])

    def drain_iq(f, q):
        ib = q % 2
        pltpu.make_async_copy(
            cat_hbm.at[f, pl.ds(0, IQ)], idx_v.at[ib], isems[ib]).wait()

    def do_field(fi, carry):
        f = lax.rem(fi + d, F)
        tcp = pltpu.async_copy(tab_hbm.at[f, d], tv, tsem)
        fire_iq(f, 0)
        tcp.wait()
        c = f * D + d
        cps = [None, None]
        for j in range(NCHUNK):
            q, jq = divmod(j, NCHUNK // NQ)
            obuf = j % 2
            ib = q % 2
            if jq == 0:
                if q + 1 < NQ:
                    fire_iq(f, q + 1)
                drain_iq(f, q)
            if cps[obuf] is not None:
                cps[obuf].wait()

            @plsc.parallel_loop(0, CHUNK // 16, unroll=UNROLL)
            def gath(i):
                vidx = idx_v[ib, pl.ds(jq * CHUNK + i * 16, 16)]
                out_v[obuf, pl.ds(i * 16, 16)] = plsc.load_gather(tv, [vidx])
            cps[obuf] = pltpu.async_copy(
                out_v.at[obuf], out_hbm.at[c, pl.ds(j * CHUNK, CHUNK)],
                osems[obuf])
        cps[0].wait()
        cps[1].wait()
        return carry

    lax.fori_loop(0, F, do_field, 0)


def kernel(categorical_features, tables):
    catT = categorical_features.T
    tabT = tables.transpose(0, 2, 1)
    mesh = plsc.VectorSubcoreMesh(core_axis_name="c", subcore_axis_name="s")
    outT = pl.kernel(
        _body,
        mesh=mesh,
        compiler_params=pltpu.CompilerParams(needs_layout_passes=False),
        out_type=jax.ShapeDtypeStruct((F * D, B), jnp.float32),
        scratch_types=[
            pltpu.VMEM((V,), jnp.float32),
            pltpu.VMEM((2, IQ), jnp.int32),
            pltpu.VMEM((2, CHUNK), jnp.float32),
            pltpu.VMEM_SHARED((NSTAGE, B), jnp.int32),
            pltpu.SemaphoreType.DMA,
            pltpu.SemaphoreType.DMA,
            pltpu.SemaphoreType.DMA,
            pltpu.SemaphoreType.DMA,
            pltpu.SemaphoreType.DMA,
        ],
    )(catT, tabT)
    return outT.T


# speedup vs baseline: 1.0010x; 1.0009x over previous
import jax
import jax.numpy as jnp
from jax import lax
from jax.experimental import pallas as pl
from jax.experimental.pallas import tpu as pltpu
from jax.experimental.pallas import tpu_sc as plsc

B = 16384
F = 26
V = 100001
D = 32

_info = plsc.get_sparse_core_info()
NC, NS = _info.num_cores, _info.num_subcores
NW = NC * NS
HALF = B // 2


CHUNK = 2048
NCHUNK = B // CHUNK
UNROLL = 8
IQ = 4096
NQ = B // IQ
NSTAGE = 16


def _body(cat_hbm, tab_hbm, out_hbm, tv, idx_v, out_v, idx_sh,
          tsem, isem0, isem1, osem0, osem1):
    s = lax.axis_index("s")
    d = s * NC + lax.axis_index("c")
    osems = [osem0, osem1]
    isems = [isem0, isem1]

    pltpu.sync_copy(cat_hbm.at[s], idx_sh.at[s])
    plsc.subcore_barrier()

    def fire_iq(f, q):
        ib = q % 2

        @pl.when(f < NSTAGE)
        def _():
            pltpu.async_copy(
                idx_sh.at[f, pl.ds(q * IQ, IQ)], idx_v.at[ib], isems[ib])

        @pl.when(f >= NSTAGE)
        def _():
            pltpu.async_copy(
                cat_hbm.at[f, pl.ds(q * IQ, IQ)], idx_v.at[ib], isems[ib])

    def drain_iq(f, q):
        ib = q % 2
        pltpu.make_async_copy(
            cat_hbm.at[f, pl.ds(0, IQ)], idx_v.at[ib], isems[ib]).wait()

    def do_field(fi, carry):
        f = lax.rem(fi + d, F)
        tcp = pltpu.async_copy(tab_hbm.at[f, d], tv, tsem)
        fire_iq(f, 0)
        tcp.wait()
        c = f * D + d
        cps = [None, None]
        for j in range(NCHUNK):
            q, jq = divmod(j, NCHUNK // NQ)
            obuf = j % 2
            ib = q % 2
            if jq == 0:
                if q + 1 < NQ:
                    fire_iq(f, q + 1)
                drain_iq(f, q)
            if cps[obuf] is not None:
                cps[obuf].wait()

            @plsc.parallel_loop(0, CHUNK // 16, unroll=UNROLL)
            def gath(i):
                vidx = idx_v[ib, pl.ds(jq * CHUNK + i * 16, 16)]
                out_v[obuf, pl.ds(i * 16, 16)] = plsc.load_gather(tv, [vidx])
            cps[obuf] = pltpu.async_copy(
                out_v.at[obuf], out_hbm.at[c, pl.ds(j * CHUNK, CHUNK)],
                osems[obuf])
        cps[0].wait()
        cps[1].wait()
        return carry

    lax.fori_loop(0, F, do_field, 0)


def kernel(categorical_features, tables):
    catT = categorical_features.T
    tabT = tables.transpose(0, 2, 1)
    mesh = plsc.VectorSubcoreMesh(core_axis_name="c", subcore_axis_name="s")
    outT = pl.kernel(
        _body,
        mesh=mesh,
        compiler_params=pltpu.CompilerParams(needs_layout_passes=False),
        out_type=jax.ShapeDtypeStruct((F * D, B), jnp.float32),
        scratch_types=[
            pltpu.VMEM((V,), jnp.float32),
            pltpu.VMEM((2, IQ), jnp.int32),
            pltpu.VMEM((2, CHUNK), jnp.float32),
            pltpu.VMEM_SHARED((NSTAGE, B), jnp.int32),
            pltpu.SemaphoreType.DMA,
            pltpu.SemaphoreType.DMA,
            pltpu.SemaphoreType.DMA,
            pltpu.SemaphoreType.DMA,
            pltpu.SemaphoreType.DMA,
        ],
    )(catT, tabT)
    return outT.T


# speedup vs baseline: 1.0014x; 1.0004x over previous
import jax
import jax.numpy as jnp
from jax import lax
from jax.experimental import pallas as pl
from jax.experimental.pallas import tpu as pltpu
from jax.experimental.pallas import tpu_sc as plsc

B = 16384
F = 26
V = 100001
D = 32

_info = plsc.get_sparse_core_info()
NC, NS = _info.num_cores, _info.num_subcores
NW = NC * NS

CHUNK = 2048
NCHUNK = B // CHUNK
UNROLL = 8
IQ = 4096
NQ = B // IQ
NSTAGE = 16


def _body(cat_hbm, tab_hbm, out_hbm, tv, idx_v, out_v, idx_sh,
          tsem, isem0, isem1, osem0, osem1):
    s = lax.axis_index("s")
    d = s * NC + lax.axis_index("c")
    osems = [osem0, osem1]
    isems = [isem0, isem1]

    pltpu.sync_copy(cat_hbm.at[s], idx_sh.at[s])
    plsc.subcore_barrier()

    def fire_iq(f, q):
        ib = q % 2

        @pl.when(f < NSTAGE)
        def _():
            pltpu.async_copy(
                idx_sh.at[f, pl.ds(q * IQ, IQ)], idx_v.at[ib], isems[ib])

        @pl.when(f >= NSTAGE)
        def _():
            pltpu.async_copy(
                cat_hbm.at[f, pl.ds(q * IQ, IQ)], idx_v.at[ib], isems[ib])

    def drain_iq(f, q):
        ib = q % 2
        pltpu.make_async_copy(
            cat_hbm.at[f, pl.ds(0, IQ)], idx_v.at[ib], isems[ib]).wait()

    def do_field(fi, carry):
        f = lax.rem(fi + d, F)
        tcp = pltpu.async_copy(tab_hbm.at[f, d], tv, tsem)
        fire_iq(f, 0)
        tcp.wait()
        c = f * D + d
        cps = [None, None]
        for j in range(NCHUNK):
            q, jq = divmod(j, NCHUNK // NQ)
            obuf = j % 2
            ib = q % 2
            if jq == 0:
                if q + 1 < NQ:
                    fire_iq(f, q + 1)
                drain_iq(f, q)
            if cps[obuf] is not None:
                cps[obuf].wait()

            @plsc.parallel_loop(0, CHUNK // 16, unroll=UNROLL)
            def gath(i):
                vidx = idx_v[ib, pl.ds(jq * CHUNK + i * 16, 16)]
                out_v[obuf, pl.ds(i * 16, 16)] = plsc.load_gather(tv, [vidx])
            cps[obuf] = pltpu.async_copy(
                out_v.at[obuf], out_hbm.at[c, pl.ds(j * CHUNK, CHUNK)],
                osems[obuf])
        cps[0].wait()
        cps[1].wait()
        return carry

    lax.fori_loop(0, F, do_field, 0)


def kernel(categorical_features, tables):
    catT = categorical_features.T
    tabT = tables.transpose(0, 2, 1)
    mesh = plsc.VectorSubcoreMesh(core_axis_name="c", subcore_axis_name="s")
    outT = pl.kernel(
        _body,
        mesh=mesh,
        compiler_params=pltpu.CompilerParams(needs_layout_passes=False),
        out_type=jax.ShapeDtypeStruct((F * D, B), jnp.float32),
        scratch_types=[
            pltpu.VMEM((V,), jnp.float32),
            pltpu.VMEM((2, IQ), jnp.int32),
            pltpu.VMEM((2, CHUNK), jnp.float32),
            pltpu.VMEM_SHARED((NSTAGE, B), jnp.int32),
            pltpu.SemaphoreType.DMA,
            pltpu.SemaphoreType.DMA,
            pltpu.SemaphoreType.DMA,
            pltpu.SemaphoreType.DMA,
            pltpu.SemaphoreType.DMA,
        ],
    )(catT, tabT)
    return outT.T
